# baseline (speedup 1.0000x reference)
import jax
import jax.numpy as jnp
from jax import lax
from jax.experimental import pallas as pl
from jax.experimental.pallas import tpu as pltpu
from jax.experimental.pallas import tpu_sc as plsc

NC = 2
NS = 16
NW = NC * NS


def _sc_gather(table, gidx, H, ebase0, ce):
    per_w = ce // NW
    n_blocks = per_w // 1024

    def body(table_hbm, gidx_hbm, out_hbm, idx_v, rows_v, sem):
        c = lax.axis_index("c")
        s = lax.axis_index("s")
        wid = s * NC + c
        base = wid * per_w

        def block(i, carry):
            pltpu.sync_copy(gidx_hbm.at[(ebase0 + base) // 1024 + i], idx_v)
            for h in range(2):
                cps = [
                    pltpu.async_copy(
                        table_hbm.at[idx_v.at[h * 4 + j]],
                        rows_v.at[pl.ds(j * 128, 128)],
                        sem,
                    )
                    for j in range(4)
                ]
                for cp in cps:
                    cp.wait()
                pltpu.sync_copy(
                    rows_v, out_hbm.at[pl.ds(base + i * 1024 + h * 512, 512)]
                )
            return carry

        lax.fori_loop(0, n_blocks, block, 0)

    return pl.kernel(
        body,
        out_type=jax.ShapeDtypeStruct((ce, H), jnp.float32),
        mesh=plsc.VectorSubcoreMesh(
            core_axis_name="c", subcore_axis_name="s",
            num_cores=NC, num_subcores=NS,
        ),
        scratch_types=[
            pltpu.VMEM((8, 128), jnp.int32),
            pltpu.VMEM((512, H), jnp.float32),
            pltpu.SemaphoreType.DMA,
        ],
    )(table, gidx)


def _sc_scatter(messages, sidx, zeros, GBATCH, A, E, H, ebase0):
    per_tile_out = A // NS
    per_tile_e = E // NS
    n_blocks = per_tile_e // 1024
    n_groups = GBATCH // NC

    def body(msg_hbm, sidx_hbm, zer_hbm, agg_hbm, idx_v, msg_v, acc_sh, sem):
        c = lax.axis_index("c")
        s = lax.axis_index("s")
        for g in range(n_groups):
            b0 = c * n_groups + g
            pltpu.sync_copy(zer_hbm, acc_sh.at[pl.ds(s * per_tile_out, per_tile_out)])
            plsc.subcore_barrier()
            ebase = b0 * E + s * per_tile_e

            def block(i, carry):
                pltpu.sync_copy(sidx_hbm.at[(ebase0 + ebase) // 1024 + i], idx_v)
                for h in range(2):
                    pltpu.sync_copy(
                        msg_hbm.at[pl.ds(ebase + i * 1024 + h * 512, 512)],
                        msg_v,
                    )
                    for j in range(4):
                        pltpu.sync_copy(
                            msg_v.at[pl.ds(j * 128, 128)],
                            acc_sh.at[idx_v.at[h * 4 + j]],
                            add=True,
                        )
                return carry

            lax.fori_loop(0, n_blocks, block, 0)
            plsc.subcore_barrier()
            pltpu.sync_copy(
                acc_sh.at[pl.ds(s * per_tile_out, per_tile_out)],
                agg_hbm.at[pl.ds(b0 * A + s * per_tile_out, per_tile_out)],
            )

    return pl.kernel(
        body,
        out_type=jax.ShapeDtypeStruct((GBATCH * A, H), jnp.float32),
        mesh=plsc.VectorSubcoreMesh(
            core_axis_name="c", subcore_axis_name="s",
            num_cores=NC, num_subcores=NS,
        ),
        scratch_types=[
            pltpu.VMEM((8, 128), jnp.int32),
            pltpu.VMEM((512, H), jnp.float32),
            pltpu.VMEM_SHARED((A + 8, H), jnp.float32),
            pltpu.SemaphoreType.DMA,
        ],
    )(messages, sidx, zeros)


def _edge_mlp_body(n_ref, bd_ref, w1_ref, b1_ref, w2_ref, b2_ref, o_ref):
    x = jnp.concatenate([n_ref[...], bd_ref[0]], axis=-1)
    x = jnp.dot(x, w1_ref[...], preferred_element_type=jnp.float32) + b1_ref[...]
    x = jnp.maximum(x, 0.0)
    o_ref[...] = jnp.dot(x, w2_ref[...], preferred_element_type=jnp.float32) + b2_ref[...]


def _tc_edge_mlp(neighbor, bond3, W1, b1r, W2, b2r, row0):
    ce, H = neighbor.shape
    E = bond3.shape[1]
    RB = 2048
    grid = (ce // RB,)
    pe = E // RB
    full = lambda shape: pl.BlockSpec(shape, lambda i: (0, 0))
    return pl.pallas_call(
        _edge_mlp_body,
        grid=grid,
        in_specs=[
            pl.BlockSpec((RB, H), lambda i: (i, 0)),
            pl.BlockSpec((1, RB, H),
                         lambda i: ((row0 // RB + i) // pe, (row0 // RB + i) % pe, 0)),
            full(W1.shape),
            full(b1r.shape),
            full(W2.shape),
            full(b2r.shape),
        ],
        out_specs=pl.BlockSpec((RB, H), lambda i: (i, 0)),
        out_shape=jax.ShapeDtypeStruct((ce, H), jnp.float32),
    )(neighbor, bond3, W1, b1r, W2, b2r)


def _node_mlp_body(a_ref, u1_ref, ub1_ref, u2_ref, ub2_ref, o_ref):
    h = jnp.dot(a_ref[...], u1_ref[...], preferred_element_type=jnp.float32) + ub1_ref[...]
    h = jnp.maximum(h, 0.0)
    o_ref[...] = jnp.dot(h, u2_ref[...], preferred_element_type=jnp.float32) + ub2_ref[...]


def _tc_node_mlp(agg, U1, ub1r, U2, ub2r):
    N, H = agg.shape
    RB = 2048
    grid = (N // RB,)
    full = lambda shape: pl.BlockSpec(shape, lambda i: (0, 0))
    return pl.pallas_call(
        _node_mlp_body,
        grid=grid,
        in_specs=[
            pl.BlockSpec((RB, H), lambda i: (i, 0)),
            full(U1.shape),
            full(ub1r.shape),
            full(U2.shape),
            full(ub2r.shape),
        ],
        out_specs=pl.BlockSpec((RB, H), lambda i: (i, 0)),
        out_shape=jax.ShapeDtypeStruct((N, H), jnp.float32),
    )(agg, U1, ub1r, U2, ub2r)


def kernel(atom_state, bond_state, src_idx, dst_idx, batch_idx, bond_mask,
           num_atoms, W1, b1, W2, b2, U1, ub1, U2, ub2):
    B, A, H = atom_state.shape
    E = bond_state.shape[1]
    BE = B * E
    GBATCH = 2
    n_chunks = B // GBATCH
    ce = GBATCH * E

    table = atom_state.reshape(B * A, H)
    gidx = (batch_idx.astype(jnp.int32) * A + dst_idx.astype(jnp.int32))
    gidx = gidx.reshape(BE // 1024, 8, 128)
    sidx = jnp.where(bond_mask != 0.0, src_idx.astype(jnp.int32),
                     jnp.int32(A)).reshape(BE // 1024, 8, 128)
    zeros = jnp.zeros((A // NS, H), jnp.float32)
    b1r, b2r = b1.reshape(1, -1), b2.reshape(1, -1)
    ub1r, ub2r = ub1.reshape(1, -1), ub2.reshape(1, -1)

    outs = []
    for q in range(n_chunks):
        nb = _sc_gather(table, gidx, H, q * ce, ce)
        msgs = _tc_edge_mlp(nb, bond_state, W1, b1r, W2, b2r, q * ce)
        agg = _sc_scatter(msgs, sidx, zeros, GBATCH, A, E, H, q * ce)
        outs.append(_tc_node_mlp(agg, U1, ub1r, U2, ub2r))
    return jnp.concatenate(outs, axis=0).reshape(B, A, H)


# speedup vs baseline: 1.0568x; 1.0568x over previous
import jax
import jax.numpy as jnp
from jax import lax
from jax.experimental import pallas as pl
from jax.experimental.pallas import tpu as pltpu
from jax.experimental.pallas import tpu_sc as plsc

NC = 2
NS = 16
NW = NC * NS


def _sc_gather(table, gidx, H, ebase0, ce):
    per_w = ce // NW
    n_blocks = per_w // 1024

    def body(table_hbm, gidx_hbm, out_hbm, idx_v, rows_v, sem):
        c = lax.axis_index("c")
        s = lax.axis_index("s")
        wid = s * NC + c
        base = wid * per_w

        def block(i, carry):
            pltpu.sync_copy(gidx_hbm.at[(ebase0 + base) // 1024 + i], idx_v)
            for h in range(2):
                cps = [
                    pltpu.async_copy(
                        table_hbm.at[idx_v.at[h * 4 + j]],
                        rows_v.at[pl.ds(j * 128, 128)],
                        sem,
                    )
                    for j in range(4)
                ]
                for cp in cps:
                    cp.wait()
                pltpu.sync_copy(
                    rows_v, out_hbm.at[pl.ds(base + i * 1024 + h * 512, 512)]
                )
            return carry

        lax.fori_loop(0, n_blocks, block, 0)

    return pl.kernel(
        body,
        out_type=jax.ShapeDtypeStruct((ce, H), jnp.float32),
        mesh=plsc.VectorSubcoreMesh(
            core_axis_name="c", subcore_axis_name="s",
            num_cores=NC, num_subcores=NS,
        ),
        scratch_types=[
            pltpu.VMEM((8, 128), jnp.int32),
            pltpu.VMEM((512, H), jnp.float32),
            pltpu.SemaphoreType.DMA,
        ],
    )(table, gidx)


def _sc_scatter(messages, sidx, zeros, GBATCH, A, E, H, ebase0):
    per_tile_out = A // NS
    per_tile_e = E // NS
    n_blocks = per_tile_e // 1024
    n_groups = GBATCH // NC

    def body(msg_hbm, sidx_hbm, zer_hbm, agg_hbm, idx_v, msg_v, acc_sh, sem):
        c = lax.axis_index("c")
        s = lax.axis_index("s")
        for g in range(n_groups):
            b0 = c * n_groups + g
            pltpu.sync_copy(zer_hbm, acc_sh.at[pl.ds(s * per_tile_out, per_tile_out)])
            plsc.subcore_barrier()
            ebase = b0 * E + s * per_tile_e

            def block(i, carry):
                pltpu.sync_copy(sidx_hbm.at[(ebase0 + ebase) // 1024 + i], idx_v)
                for h in range(2):
                    pltpu.sync_copy(
                        msg_hbm.at[pl.ds(ebase + i * 1024 + h * 512, 512)],
                        msg_v,
                    )
                    for j in range(4):
                        pltpu.sync_copy(
                            msg_v.at[pl.ds(j * 128, 128)],
                            acc_sh.at[idx_v.at[h * 4 + j]],
                            add=True,
                        )
                return carry

            lax.fori_loop(0, n_blocks, block, 0)
            plsc.subcore_barrier()
            pltpu.sync_copy(
                acc_sh.at[pl.ds(s * per_tile_out, per_tile_out)],
                agg_hbm.at[pl.ds(b0 * A + s * per_tile_out, per_tile_out)],
            )

    return pl.kernel(
        body,
        out_type=jax.ShapeDtypeStruct((GBATCH * A, H), jnp.float32),
        mesh=plsc.VectorSubcoreMesh(
            core_axis_name="c", subcore_axis_name="s",
            num_cores=NC, num_subcores=NS,
        ),
        scratch_types=[
            pltpu.VMEM((8, 128), jnp.int32),
            pltpu.VMEM((512, H), jnp.float32),
            pltpu.VMEM_SHARED((A + 8, H), jnp.float32),
            pltpu.SemaphoreType.DMA,
        ],
    )(messages, sidx, zeros)


def _edge_mlp_body(n_ref, bd_ref, w1_ref, b1_ref, w2_ref, b2_ref, o_ref):
    x = jnp.concatenate([n_ref[...], bd_ref[0]], axis=-1)
    x = jnp.dot(x, w1_ref[...], preferred_element_type=jnp.float32) + b1_ref[...]
    x = jnp.maximum(x, 0.0)
    o_ref[...] = jnp.dot(x, w2_ref[...], preferred_element_type=jnp.float32) + b2_ref[...]


def _tc_edge_mlp(neighbor, bond3, W1, b1r, W2, b2r, row0):
    ce, H = neighbor.shape
    E = bond3.shape[1]
    RB = 2048
    grid = (ce // RB,)
    pe = E // RB
    full = lambda shape: pl.BlockSpec(shape, lambda i: (0, 0))
    return pl.pallas_call(
        _edge_mlp_body,
        grid=grid,
        in_specs=[
            pl.BlockSpec((RB, H), lambda i: (i, 0)),
            pl.BlockSpec((1, RB, H),
                         lambda i: ((row0 // RB + i) // pe, (row0 // RB + i) % pe, 0)),
            full(W1.shape),
            full(b1r.shape),
            full(W2.shape),
            full(b2r.shape),
        ],
        out_specs=pl.BlockSpec((RB, H), lambda i: (i, 0)),
        out_shape=jax.ShapeDtypeStruct((ce, H), jnp.float32),
    )(neighbor, bond3, W1, b1r, W2, b2r)


def _node_mlp_body(a_ref, u1_ref, ub1_ref, u2_ref, ub2_ref, o_ref):
    h = jnp.dot(a_ref[...], u1_ref[...], preferred_element_type=jnp.float32) + ub1_ref[...]
    h = jnp.maximum(h, 0.0)
    o_ref[...] = jnp.dot(h, u2_ref[...], preferred_element_type=jnp.float32) + ub2_ref[...]


def _tc_node_mlp(agg, U1, ub1r, U2, ub2r):
    N, H = agg.shape
    RB = 2048
    grid = (N // RB,)
    full = lambda shape: pl.BlockSpec(shape, lambda i: (0, 0))
    return pl.pallas_call(
        _node_mlp_body,
        grid=grid,
        in_specs=[
            pl.BlockSpec((RB, H), lambda i: (i, 0)),
            full(U1.shape),
            full(ub1r.shape),
            full(U2.shape),
            full(ub2r.shape),
        ],
        out_specs=pl.BlockSpec((RB, H), lambda i: (i, 0)),
        out_shape=jax.ShapeDtypeStruct((N, H), jnp.float32),
    )(agg, U1, ub1r, U2, ub2r)


def kernel(atom_state, bond_state, src_idx, dst_idx, batch_idx, bond_mask,
           num_atoms, W1, b1, W2, b2, U1, ub1, U2, ub2):
    B, A, H = atom_state.shape
    E = bond_state.shape[1]
    BE = B * E
    bounds = [0, 2, 6, 10, 14, 16]

    table = atom_state.reshape(B * A, H)
    gidx = (batch_idx.astype(jnp.int32) * A + dst_idx.astype(jnp.int32))
    gidx = gidx.reshape(BE // 1024, 8, 128)
    sidx = jnp.where(bond_mask != 0.0, src_idx.astype(jnp.int32),
                     jnp.int32(A)).reshape(BE // 1024, 8, 128)
    zeros = jnp.zeros((A // NS, H), jnp.float32)
    b1r, b2r = b1.reshape(1, -1), b2.reshape(1, -1)
    ub1r, ub2r = ub1.reshape(1, -1), ub2.reshape(1, -1)

    outs = []
    for b0, b1 in zip(bounds[:-1], bounds[1:]):
        ce = (b1 - b0) * E
        nb = _sc_gather(table, gidx, H, b0 * E, ce)
        msgs = _tc_edge_mlp(nb, bond_state, W1, b1r, W2, b2r, b0 * E)
        agg = _sc_scatter(msgs, sidx, zeros, b1 - b0, A, E, H, b0 * E)
        outs.append(_tc_node_mlp(agg, U1, ub1r, U2, ub2r))
    return jnp.concatenate(outs, axis=0).reshape(B, A, H)
